# baseline (speedup 1.0000x reference)
import functools

import jax
import jax.numpy as jnp
from jax.experimental import pallas as pl
from jax.experimental.pallas import tpu as pltpu

B, D = 16384, 64
TILE = 2048


def _fused_kernel(x_ref, h_ref, w_ref, u_ref, wb_ref, ub_ref, qrb_ref, qlb_ref,
                  g_ref, b_ref, o_ref):
    x = x_ref[:, :D]
    h = h_ref[:, :D]
    pre = jnp.dot(x, w_ref[...], preferred_element_type=jnp.float32)
    pre = pre + jnp.dot(h, u_ref[...], preferred_element_type=jnp.float32)
    pre = pre + (wb_ref[...] + ub_ref[...] + qrb_ref[...] + qlb_ref[...])
    mu = jnp.mean(pre, axis=-1, keepdims=True)
    cent = pre - mu
    var = jnp.mean(cent * cent, axis=-1, keepdims=True)
    normed = cent * jax.lax.rsqrt(var + 1e-5) * g_ref[...] + b_ref[...]
    o_ref[:, :D] = jax.nn.sigmoid(normed)


@functools.partial(jax.jit, static_argnames=("interpret",))
def _run(x, h_prev, W_w, U_w, W_b, U_b, Qr_b, Ql_b, ln_g, ln_b, interpret=False):
    grid = (B // TILE,)
    wide_spec = pl.BlockSpec((TILE, 128), lambda i: (i, 0))
    full_spec = pl.BlockSpec((D, D), lambda i: (0, 0))
    vec_spec = pl.BlockSpec((1, D), lambda i: (0, 0))
    return pl.pallas_call(
        _fused_kernel,
        grid=grid,
        in_specs=[wide_spec, wide_spec, full_spec, full_spec,
                  vec_spec, vec_spec, vec_spec, vec_spec, vec_spec, vec_spec],
        out_specs=wide_spec,
        out_shape=jax.ShapeDtypeStruct((B, D), jnp.float32),
        compiler_params=pltpu.CompilerParams(dimension_semantics=("parallel",)),
        interpret=interpret,
    )(x, h_prev, W_w, U_w, W_b, U_b, Qr_b, Ql_b, ln_g, ln_b)


def kernel(x, h_prev, W_w, W_b, U_w, U_b, M_w, M_b, Qr_w, Qr_b, Ql_w, Ql_b, ln_g, ln_b):
    r = lambda v: v.reshape(1, D)
    return _run(x, h_prev, W_w, U_w, r(W_b), r(U_b), r(Qr_b), r(Ql_b), r(ln_g), r(ln_b))
